# baseline (speedup 1.0000x reference)
import jax
import jax.numpy as jnp
from jax.experimental import pallas as pl
from jax.experimental.pallas import tpu as pltpu

E = 8
H = 1024
FF = 3584
CF = 1792
NF = FF // CF


def _dot_nt(a, b):
    return jax.lax.dot_general(
        a, b, (((1,), (1,)), ((), ())), preferred_element_type=jnp.float32
    )


def _moe_body(x_ref, gate_ref, w1_ref, w3_ref, w2_ref, out_ref, coef_ref):
    e = pl.program_id(0)
    f = pl.program_id(1)

    x = x_ref[...]

    @pl.when((e == 0) & (f == 0))
    def _init():
        logits = _dot_nt(x, gate_ref[...])
        cols = jax.lax.broadcasted_iota(jnp.int32, logits.shape, 1)
        m1 = jnp.max(logits, axis=1, keepdims=True)
        eq1 = logits == m1
        i1 = jnp.min(jnp.where(eq1, cols, E), axis=1, keepdims=True)
        mask1 = cols == i1
        rest = jnp.where(mask1, -jnp.inf, logits)
        m2 = jnp.max(rest, axis=1, keepdims=True)
        eq2 = rest == m2
        i2 = jnp.min(jnp.where(eq2, cols, E), axis=1, keepdims=True)
        mask2 = cols == i2
        coef = jnp.where(
            mask1 | mask2,
            jnp.exp(logits - m1) / (1.0 + jnp.exp(m2 - m1)),
            0.0,
        )
        coef_ref[...] = coef
        out_ref[...] = jnp.zeros_like(out_ref)

    a = _dot_nt(x, w1_ref[0])
    b = _dot_nt(x, w3_ref[0])
    g = a * jax.nn.sigmoid(a) * b
    cols = jax.lax.broadcasted_iota(jnp.int32, coef_ref.shape, 1)
    c = jnp.sum(jnp.where(cols == e, coef_ref[...], 0.0), axis=1, keepdims=True)
    out_ref[...] += _dot_nt(c * g, w2_ref[0])


def kernel(hidden_states, gate_w, w1, w3, w2, prefetch_expert_idx):
    b, s, h = hidden_states.shape
    t = b * s
    x = hidden_states.reshape(t, h)

    out = pl.pallas_call(
        _moe_body,
        grid=(E, NF),
        in_specs=[
            pl.BlockSpec((t, H), lambda e, f: (0, 0)),
            pl.BlockSpec((E, H), lambda e, f: (0, 0)),
            pl.BlockSpec((1, CF, H), lambda e, f: (e, f, 0)),
            pl.BlockSpec((1, CF, H), lambda e, f: (e, f, 0)),
            pl.BlockSpec((1, H, CF), lambda e, f: (e, 0, f)),
        ],
        out_specs=pl.BlockSpec((t, H), lambda e, f: (0, 0)),
        out_shape=jax.ShapeDtypeStruct((t, H), jnp.float32),
        scratch_shapes=[pltpu.VMEM((t, E), jnp.float32)],
        compiler_params=pltpu.CompilerParams(
            dimension_semantics=("arbitrary", "arbitrary"),
        ),
    )(x, gate_w, w1, w3, w2)

    return out.reshape(b, s, h)


# speedup vs baseline: 1.0068x; 1.0068x over previous
import jax
import jax.numpy as jnp
from jax.experimental import pallas as pl
from jax.experimental.pallas import tpu as pltpu

E = 8
H = 1024
FF = 3584
CF = 896
NF = FF // CF


def _dot_nt(a, b):
    return jax.lax.dot_general(
        a, b, (((1,), (1,)), ((), ())), preferred_element_type=jnp.float32
    )


def _moe_body(x_ref, gate_ref, w1_ref, w3_ref, w2_ref, out_ref, coef_ref):
    e = pl.program_id(0)
    f = pl.program_id(1)

    x = x_ref[...]

    @pl.when((e == 0) & (f == 0))
    def _init():
        logits = _dot_nt(x, gate_ref[...])
        cols = jax.lax.broadcasted_iota(jnp.int32, logits.shape, 1)
        m1 = jnp.max(logits, axis=1, keepdims=True)
        eq1 = logits == m1
        i1 = jnp.min(jnp.where(eq1, cols, E), axis=1, keepdims=True)
        mask1 = cols == i1
        rest = jnp.where(mask1, -jnp.inf, logits)
        m2 = jnp.max(rest, axis=1, keepdims=True)
        eq2 = rest == m2
        i2 = jnp.min(jnp.where(eq2, cols, E), axis=1, keepdims=True)
        mask2 = cols == i2
        coef = jnp.where(
            mask1 | mask2,
            jnp.exp(logits - m1) / (1.0 + jnp.exp(m2 - m1)),
            0.0,
        )
        coef_ref[...] = coef
        out_ref[...] = jnp.zeros_like(out_ref)

    a = _dot_nt(x, w1_ref[0])
    b = _dot_nt(x, w3_ref[0])
    g = a * jax.nn.sigmoid(a) * b
    cols = jax.lax.broadcasted_iota(jnp.int32, coef_ref.shape, 1)
    c = jnp.sum(jnp.where(cols == e, coef_ref[...], 0.0), axis=1, keepdims=True)
    out_ref[...] += _dot_nt(c * g, w2_ref[0])


def kernel(hidden_states, gate_w, w1, w3, w2, prefetch_expert_idx):
    b, s, h = hidden_states.shape
    t = b * s
    x = hidden_states.reshape(t, h)

    out = pl.pallas_call(
        _moe_body,
        grid=(E, NF),
        in_specs=[
            pl.BlockSpec((t, H), lambda e, f: (0, 0)),
            pl.BlockSpec((E, H), lambda e, f: (0, 0)),
            pl.BlockSpec((1, CF, H), lambda e, f: (e, f, 0)),
            pl.BlockSpec((1, CF, H), lambda e, f: (e, f, 0)),
            pl.BlockSpec((1, H, CF), lambda e, f: (e, 0, f)),
        ],
        out_specs=pl.BlockSpec((t, H), lambda e, f: (0, 0)),
        out_shape=jax.ShapeDtypeStruct((t, H), jnp.float32),
        scratch_shapes=[pltpu.VMEM((t, E), jnp.float32)],
        compiler_params=pltpu.CompilerParams(
            dimension_semantics=("arbitrary", "arbitrary"),
        ),
    )(x, gate_w, w1, w3, w2)

    return out.reshape(b, s, h)
